# Initial kernel scaffold; baseline (speedup 1.0000x reference)
import jax
import jax.numpy as jnp
from jax.experimental import pallas as pl


def kernel(locs, W, b):
    raise NotImplementedError("write your pallas kernel here")


# speedup vs baseline: 28.8229x; 28.8229x over previous
import functools

import jax
import jax.numpy as jnp
from jax.experimental import pallas as pl
from jax.experimental.pallas import tpu as pltpu

_K = 16
_N = 512
_D = 128
_FPAD = 40


def _body(locsT_ref, locs_ref, Wp_ref, b_ref, out_ref, feats_ref):
    x_row = locsT_ref[0, 0:1, :]
    y_row = locsT_ref[0, 1:2, :]
    x_col = locs_ref[0, :, 0:1]
    y_col = locs_ref[0, :, 1:2]
    dxm = x_row - x_col
    dym = y_row - y_col
    d2 = dxm * dxm + dym * dym
    colj = jax.lax.broadcasted_iota(jnp.int32, (_N, _N), 1)
    rowi = jax.lax.broadcasted_iota(jnp.int32, (_N, _N), 0)
    inf = jnp.float32(jnp.inf)
    d2 = jnp.where(rowi == colj, inf, d2)

    feats_ref[:, 0:2] = locs_ref[0]
    feats_ref[:, 34:_FPAD] = jnp.zeros((_N, _FPAD - 34), jnp.float32)
    for k in range(_K):
        mind2 = jnp.min(d2, axis=1, keepdims=True)
        mask = d2 == mind2
        feats_ref[:, 2 + k:3 + k] = jnp.sum(
            jnp.where(mask, dxm, 0.0), axis=1, keepdims=True)
        feats_ref[:, 18 + k:19 + k] = jnp.sum(
            jnp.where(mask, dym, 0.0), axis=1, keepdims=True)
        d2 = jnp.where(mask, inf, d2)

    out_ref[0] = (
        jnp.dot(feats_ref[...], Wp_ref[...], preferred_element_type=jnp.float32)
        + b_ref[...]
    )


@jax.jit
def kernel(locs, W, b):
    B, N, _ = locs.shape
    locsT = locs.transpose(0, 2, 1)
    order = [0, 1] + [2 + 2 * k for k in range(_K)] + [3 + 2 * k for k in range(_K)]
    Wp = jnp.zeros((_FPAD, _D), W.dtype).at[:34].set(W[jnp.asarray(order)])
    b2 = b.reshape(1, _D)
    out = pl.pallas_call(
        _body,
        grid=(B,),
        in_specs=[
            pl.BlockSpec((1, 2, N), lambda i: (i, 0, 0)),
            pl.BlockSpec((1, N, 2), lambda i: (i, 0, 0)),
            pl.BlockSpec((_FPAD, _D), lambda i: (0, 0)),
            pl.BlockSpec((1, _D), lambda i: (0, 0)),
        ],
        out_specs=pl.BlockSpec((1, N, _D), lambda i: (i, 0, 0)),
        out_shape=jax.ShapeDtypeStruct((B, N, _D), jnp.float32),
        scratch_shapes=[pltpu.VMEM((N, _FPAD), jnp.float32)],
    )(locsT, locs, Wp, b2)
    return out
